# baseline (speedup 1.0000x reference)
import functools

import jax
import jax.numpy as jnp
from jax import lax
from jax.experimental import pallas as pl
from jax.experimental.pallas import tpu as pltpu
from jax.experimental.pallas import tpu_sc as plsc

_NUM_EMBEDDINGS = 1000000
_DIM = 32
_BATCH = 16384
_FIELDS = 26

_NC = 2
_NS = 16
_NW = _NC * _NS
_R_PER_W = _BATCH // _NW
_RG = 16
_N_GROUPS = _R_PER_W // _RG


@functools.cache
def _build_gather_kernel():
    mesh = plsc.VectorSubcoreMesh(
        core_axis_name="c", subcore_axis_name="s", num_cores=_NC, num_subcores=_NS
    )

    @functools.partial(
        pl.kernel,
        mesh=mesh,
        out_type=jax.ShapeDtypeStruct((_BATCH, _FIELDS, _DIM), jnp.float32),
        scratch_types=[
            pltpu.VMEM((_R_PER_W, _FIELDS), jnp.int32),
            pltpu.VMEM((_RG, _FIELDS, _DIM), jnp.float32),
            pltpu.SemaphoreType.DMA,
        ],
        compiler_params=pltpu.CompilerParams(use_tc_tiling_on_sc=False),
    )
    def gather_kernel(x_hbm, table_hbm, out_hbm, idx_v, rows_v, sem):
        wid = lax.axis_index("s") * _NC + lax.axis_index("c")
        row0 = wid * _R_PER_W
        pltpu.sync_copy(x_hbm.at[pl.ds(row0, _R_PER_W)], idx_v)

        def body(g, carry):
            r0 = g * _RG
            handles = [
                pltpu.async_copy(
                    table_hbm.at[idx_v.at[r0 + r]],
                    rows_v.at[r],
                    sem,
                )
                for r in range(_RG)
            ]
            for h in handles:
                h.wait()
            pltpu.sync_copy(rows_v, out_hbm.at[pl.ds(row0 + r0, _RG)])
            return carry

        lax.fori_loop(0, _N_GROUPS, body, 0)

    return gather_kernel


def kernel(x, embedding_weight):
    return _build_gather_kernel()(x.astype(jnp.int32), embedding_weight)


# speedup vs baseline: 1.2503x; 1.2503x over previous
import functools

import jax
import jax.numpy as jnp
from jax import lax
from jax.experimental import pallas as pl
from jax.experimental.pallas import tpu as pltpu
from jax.experimental.pallas import tpu_sc as plsc

_NUM_EMB = 1000000
_DIM = 32
_BATCH = 16384
_FIELDS = 26

_NC = 2
_NS = 16
_NW = _NC * _NS

_CBLK = 8192
_QBLK = _CBLK // 4
_GRID = (_NUM_EMB + _CBLK - 1) // _CBLK
_S_ROWS = _GRID * _QBLK


def _detile_body(tt_ref, s_ref):
    t = jnp.transpose(tt_ref[...])
    parts = [t[a * _QBLK:(a + 1) * _QBLK, :] for a in range(4)]
    s_ref[...] = jnp.concatenate(parts, axis=1)


@functools.cache
def _build_detile():
    return pl.pallas_call(
        _detile_body,
        grid=(_GRID,),
        in_specs=[pl.BlockSpec((_DIM, _CBLK), lambda g: (0, g))],
        out_specs=pl.BlockSpec((_QBLK, 128), lambda g: (g, 0)),
        out_shape=jax.ShapeDtypeStruct((_S_ROWS, 128), jnp.float32),
    )


@functools.cache
def _build_gather():
    mesh = plsc.VectorSubcoreMesh(
        core_axis_name="c", subcore_axis_name="s", num_cores=_NC, num_subcores=_NS
    )

    @functools.partial(
        pl.kernel,
        mesh=mesh,
        out_type=jax.ShapeDtypeStruct((_FIELDS * 4 * 128 * 8, 128), jnp.float32),
        scratch_types=[
            pltpu.VMEM((_FIELDS, 4, 128), jnp.int32),
            pltpu.VMEM((128, _DIM), jnp.float32),
            pltpu.VMEM((_DIM, 128), jnp.float32),
            pltpu.SemaphoreType.DMA,
        ],
        compiler_params=pltpu.CompilerParams(
            use_tc_tiling_on_sc=False, needs_layout_passes=False
        ),
    )
    def gather_kernel(xt3_hbm, s_hbm, o_hbm, idx_v, rows_v, tile_v, sem):
        wid = lax.axis_index("s") * _NC + lax.axis_index("c")
        bt0 = wid * 4
        lane = lax.iota(jnp.int32, 16)
        pltpu.sync_copy(xt3_hbm.at[:, pl.ds(bt0, 4)], idx_v)

        def tbody(fb, carry):
            f = fb // 4
            b = fb % 4
            for g in range(8):
                v = idx_v[f, b, pl.ds(g * 16, 16)]
                j = ((v >> 13) << 13) | ((v & 2047) << 2) | ((v >> 11) & 3)
                idx_v[f, b, pl.ds(g * 16, 16)] = j
            return carry

        lax.fori_loop(0, _FIELDS * 4, tbody, 0)

        def unit(u, carry):
            b = u // _FIELDS
            f = u % _FIELDS
            pltpu.async_copy(s_hbm.at[idx_v.at[f, b]], rows_v, sem).wait()

            def rbody(r, c2):
                for half in range(2):
                    v = rows_v[r, pl.ds(half * 16, 16)]
                    plsc.store_scatter(
                        tile_v,
                        [lane + half * 16, jnp.full((16,), r, jnp.int32)],
                        v,
                    )
                return c2

            lax.fori_loop(0, 128, rbody, 0)

            bt = bt0 + b
            for st in range(4):
                q = (f * 4 + st) * 128 + bt
                pltpu.sync_copy(
                    tile_v.at[pl.ds(st * 8, 8)], o_hbm.at[pl.ds(q * 8, 8)]
                )
            return carry

        lax.fori_loop(0, 4 * _FIELDS, unit, 0)

    return gather_kernel


def kernel(x, embedding_weight):
    tt = jnp.transpose(embedding_weight)
    s = _build_detile()(tt)
    s_r = s.reshape(_S_ROWS * 4, _DIM)
    xt3 = jnp.transpose(x.astype(jnp.int32)).reshape(_FIELDS, 128, 128)
    o = _build_gather()(xt3, s_r)
    o5 = o.reshape(_FIELDS, 4, 128, 8, 128)
    o5t = jnp.transpose(o5, (2, 4, 0, 1, 3))
    return o5t.reshape(_BATCH, _FIELDS, _DIM)


# speedup vs baseline: 1.4426x; 1.1538x over previous
import functools

import jax
import jax.numpy as jnp
from jax import lax
from jax.experimental import pallas as pl
from jax.experimental.pallas import tpu as pltpu
from jax.experimental.pallas import tpu_sc as plsc

_NUM_EMB = 1000000
_DIM = 32
_BATCH = 16384
_FIELDS = 26

_NC = 2
_NS = 16

_LOG_C = 14
_CBLK = 1 << _LOG_C
_LOG_Q = _LOG_C - 2
_QBLK = _CBLK // 4
_GRID = (_NUM_EMB + _CBLK - 1) // _CBLK
_S_ROWS = _GRID * _QBLK


def _detile_body(tt_ref, s_ref):
    t = jnp.transpose(tt_ref[...])
    parts = [t[a * _QBLK:(a + 1) * _QBLK, :] for a in range(4)]
    s_ref[...] = jnp.concatenate(parts, axis=1)


@functools.cache
def _build_detile():
    return pl.pallas_call(
        _detile_body,
        grid=(_GRID,),
        in_specs=[pl.BlockSpec((_DIM, _CBLK), lambda g: (0, g))],
        out_specs=pl.BlockSpec((_QBLK, 128), lambda g: (g, 0)),
        out_shape=jax.ShapeDtypeStruct((_S_ROWS, 128), jnp.float32),
    )


_N_UNITS = 4 * _FIELDS


@functools.cache
def _build_gather():
    mesh = plsc.VectorSubcoreMesh(
        core_axis_name="c", subcore_axis_name="s", num_cores=_NC, num_subcores=_NS
    )

    @functools.partial(
        pl.kernel,
        mesh=mesh,
        out_type=jax.ShapeDtypeStruct((_FIELDS * 4 * 128 * 8, 128), jnp.float32),
        scratch_types=[
            pltpu.VMEM((_FIELDS, 4, 128), jnp.int32),
            pltpu.VMEM((2, 128, _DIM), jnp.float32),
            pltpu.VMEM((_DIM, 128), jnp.float32),
            pltpu.SemaphoreType.DMA,
            pltpu.SemaphoreType.DMA,
        ],
        compiler_params=pltpu.CompilerParams(
            use_tc_tiling_on_sc=False, needs_layout_passes=False
        ),
    )
    def gather_kernel(xt3_hbm, s_hbm, o_hbm, idx_v, rows_v, tile_v, sem0, sem1):
        wid = lax.axis_index("s") * _NC + lax.axis_index("c")
        bt0 = wid * 4
        lane = lax.iota(jnp.int32, 16)
        pltpu.sync_copy(xt3_hbm.at[:, pl.ds(bt0, 4)], idx_v)

        def tbody(fb, carry):
            f = fb // 4
            b = fb % 4
            for g in range(8):
                v = idx_v[f, b, pl.ds(g * 16, 16)]
                j = (
                    ((v >> _LOG_C) << _LOG_C)
                    | ((v & (_QBLK - 1)) << 2)
                    | ((v >> _LOG_Q) & 3)
                )
                idx_v[f, b, pl.ds(g * 16, 16)] = j
            return carry

        lax.fori_loop(0, _FIELDS * 4, tbody, 0)

        def fire(u, e, sem):
            b = u // _FIELDS
            f = u % _FIELDS
            return pltpu.async_copy(s_hbm.at[idx_v.at[f, b]], rows_v.at[e], sem)

        def drain(u, e, sem):
            b = u // _FIELDS
            f = u % _FIELDS
            pltpu.make_async_copy(
                s_hbm.at[idx_v.at[f, b]], rows_v.at[e], sem
            ).wait()

        def process(u, e):
            b = u // _FIELDS
            f = u % _FIELDS

            def rbody(rg, c2):
                for rr in range(16):
                    r = rg * 16 + rr
                    for half in range(2):
                        v = rows_v[e, r, pl.ds(half * 16, 16)]
                        plsc.store_scatter(
                            tile_v,
                            [lane + half * 16, jnp.full((16,), r, jnp.int32)],
                            v,
                        )
                return c2

            lax.fori_loop(0, 8, rbody, 0)

            bt = bt0 + b
            for st in range(4):
                q = (f * 4 + st) * 128 + bt
                pltpu.sync_copy(
                    tile_v.at[pl.ds(st * 8, 8)], o_hbm.at[pl.ds(q * 8, 8)]
                )

        fire(0, 0, sem0)
        fire(1, 1, sem1)

        def body(k, carry):
            u0 = 2 * k
            drain(u0, 0, sem0)
            process(u0, 0)

            @pl.when(u0 + 2 < _N_UNITS)
            def _():
                fire(u0 + 2, 0, sem0)

            drain(u0 + 1, 1, sem1)
            process(u0 + 1, 1)

            @pl.when(u0 + 3 < _N_UNITS)
            def _():
                fire(u0 + 3, 1, sem1)

            return carry

        lax.fori_loop(0, _N_UNITS // 2, body, 0)

    return gather_kernel


def kernel(x, embedding_weight):
    tt = jnp.transpose(embedding_weight)
    s = _build_detile()(tt)
    s_r = s.reshape(_S_ROWS * 4, _DIM)
    xt3 = jnp.transpose(x.astype(jnp.int32)).reshape(_FIELDS, 128, 128)
    o = _build_gather()(xt3, s_r)
    o5 = o.reshape(_FIELDS, 4, 128, 8, 128)
    o5t = jnp.transpose(o5, (2, 4, 0, 1, 3))
    return o5t.reshape(_BATCH, _FIELDS, _DIM)


# speedup vs baseline: 1.5467x; 1.0722x over previous
import functools

import jax
import jax.numpy as jnp
from jax import lax
from jax.experimental import pallas as pl
from jax.experimental.pallas import tpu as pltpu
from jax.experimental.pallas import tpu_sc as plsc

_NUM_EMB = 1000000
_DIM = 32
_BATCH = 16384
_FIELDS = 26

_NC = 2
_NS = 16

_LOG_C = 14
_CBLK = 1 << _LOG_C
_LOG_Q = _LOG_C - 2
_QBLK = _CBLK // 4
_GRID = (_NUM_EMB + _CBLK - 1) // _CBLK
_S_ROWS = _GRID * _QBLK


def _detile_body(tt_ref, s_ref):
    t = jnp.transpose(tt_ref[...])
    parts = [t[a * _QBLK:(a + 1) * _QBLK, :] for a in range(4)]
    s_ref[...] = jnp.concatenate(parts, axis=1)


@functools.cache
def _build_detile():
    return pl.pallas_call(
        _detile_body,
        grid=(_GRID,),
        in_specs=[pl.BlockSpec((_DIM, _CBLK), lambda g: (0, g))],
        out_specs=pl.BlockSpec((_QBLK, 128), lambda g: (g, 0)),
        out_shape=jax.ShapeDtypeStruct((_S_ROWS, 128), jnp.float32),
    )


_N_UNITS = 4 * _FIELDS


@functools.cache
def _build_gather():
    mesh = plsc.VectorSubcoreMesh(
        core_axis_name="c", subcore_axis_name="s", num_cores=_NC, num_subcores=_NS
    )

    @functools.partial(
        pl.kernel,
        mesh=mesh,
        out_type=jax.ShapeDtypeStruct((_FIELDS * 4 * 128 * 8, 128), jnp.float32),
        scratch_types=[
            pltpu.VMEM((_FIELDS, 4, 128), jnp.int32),
            pltpu.VMEM((2, 128, _DIM), jnp.float32),
            pltpu.VMEM((2, _DIM, 128), jnp.float32),
            pltpu.SemaphoreType.DMA,
            pltpu.SemaphoreType.DMA,
            pltpu.SemaphoreType.DMA,
            pltpu.SemaphoreType.DMA,
        ],
        compiler_params=pltpu.CompilerParams(
            use_tc_tiling_on_sc=False, needs_layout_passes=False
        ),
    )
    def gather_kernel(
        xt3_hbm, s_hbm, o_hbm, idx_v, rows_v, tile_v, sem0, sem1, semw0, semw1
    ):
        wid = lax.axis_index("s") * _NC + lax.axis_index("c")
        bt0 = wid * 4
        lane = lax.iota(jnp.int32, 16)
        pltpu.sync_copy(xt3_hbm.at[:, pl.ds(bt0, 4)], idx_v)

        def tbody(fb, carry):
            f = fb // 4
            b = fb % 4
            for g in range(8):
                v = idx_v[f, b, pl.ds(g * 16, 16)]
                j = (
                    ((v >> _LOG_C) << _LOG_C)
                    | ((v & (_QBLK - 1)) << 2)
                    | ((v >> _LOG_Q) & 3)
                )
                idx_v[f, b, pl.ds(g * 16, 16)] = j
            return carry

        lax.fori_loop(0, _FIELDS * 4, tbody, 0)

        def fire(u, e, sem):
            b = u // _FIELDS
            f = u % _FIELDS
            return pltpu.async_copy(s_hbm.at[idx_v.at[f, b]], rows_v.at[e], sem)

        def drain(u, e, sem):
            b = u // _FIELDS
            f = u % _FIELDS
            pltpu.make_async_copy(
                s_hbm.at[idx_v.at[f, b]], rows_v.at[e], sem
            ).wait()

        def qbase(u, st):
            b = u // _FIELDS
            f = u % _FIELDS
            return (f * 4 + st) * 128 + bt0 + b

        def drain_writes(u, e, semw):
            for st in range(4):
                q = qbase(u, st)
                pltpu.make_async_copy(
                    tile_v.at[e, pl.ds(st * 8, 8)], o_hbm.at[pl.ds(q * 8, 8)], semw
                ).wait()

        def process(u, e, semw):
            @pl.when(u >= 2)
            def _():
                drain_writes(u - 2, e, semw)

            def rbody(rg, c2):
                for rr in range(16):
                    r = rg * 16 + rr
                    for half in range(2):
                        v = rows_v[e, r, pl.ds(half * 16, 16)]
                        plsc.store_scatter(
                            tile_v.at[e],
                            [lane + half * 16, jnp.full((16,), r, jnp.int32)],
                            v,
                        )
                return c2

            lax.fori_loop(0, 8, rbody, 0)

            for st in range(4):
                q = qbase(u, st)
                pltpu.async_copy(
                    tile_v.at[e, pl.ds(st * 8, 8)], o_hbm.at[pl.ds(q * 8, 8)], semw
                )

        fire(0, 0, sem0)
        fire(1, 1, sem1)

        def body(k, carry):
            u0 = 2 * k
            drain(u0, 0, sem0)
            process(u0, 0, semw0)

            @pl.when(u0 + 2 < _N_UNITS)
            def _():
                fire(u0 + 2, 0, sem0)

            drain(u0 + 1, 1, sem1)
            process(u0 + 1, 1, semw1)

            @pl.when(u0 + 3 < _N_UNITS)
            def _():
                fire(u0 + 3, 1, sem1)

            return carry

        lax.fori_loop(0, _N_UNITS // 2, body, 0)
        drain_writes(_N_UNITS - 2, 0, semw0)
        drain_writes(_N_UNITS - 1, 1, semw1)

    return gather_kernel


def kernel(x, embedding_weight):
    tt = jnp.transpose(embedding_weight)
    s = _build_detile()(tt)
    s_r = s.reshape(_S_ROWS * 4, _DIM)
    xt3 = jnp.transpose(x.astype(jnp.int32)).reshape(_FIELDS, 128, 128)
    o = _build_gather()(xt3, s_r)
    o5 = o.reshape(_FIELDS, 4, 128, 8, 128)
    o5t = jnp.transpose(o5, (2, 4, 0, 1, 3))
    return o5t.reshape(_BATCH, _FIELDS, _DIM)
